# baseline (speedup 1.0000x reference)
import jax
import jax.numpy as jnp
from jax.experimental import pallas as pl
from jax.experimental.pallas import tpu as pltpu


def _mask_body(q_ref, x_ref, o_ref):
    q = q_ref[pl.program_id(0), 0]
    o_ref[...] = (x_ref[...] >= q).astype(jnp.float32)


def kernel(scale, pr):
    bs, ch, w, h = scale.shape
    n = ch * w * h
    p = jnp.where(pr > 10, 10, pr) * 0.1
    pr_bis = 1.0 - p
    flat = scale.reshape(bs, n)
    q = jnp.quantile(flat, pr_bis, axis=1, method="linear")
    q_eff = jnp.where(pr == 0, jnp.inf, q).astype(jnp.float32)

    x3 = flat.reshape(bs, n // 128, 128)
    out = pl.pallas_call(
        _mask_body,
        grid=(bs,),
        in_specs=[
            pl.BlockSpec((bs, 1), lambda b: (0, 0), memory_space=pltpu.SMEM),
            pl.BlockSpec((1, n // 128, 128), lambda b: (b, 0, 0)),
        ],
        out_specs=pl.BlockSpec((1, n // 128, 128), lambda b: (b, 0, 0)),
        out_shape=jax.ShapeDtypeStruct((bs, n // 128, 128), jnp.float32),
    )(q_eff.reshape(bs, 1), x3)
    return out.reshape(bs, ch, w, h)


# speedup vs baseline: 8.8952x; 8.8952x over previous
import functools

import jax
import jax.numpy as jnp
from jax import lax
from jax.experimental import pallas as pl
from jax.experimental.pallas import tpu as pltpu
from jax.experimental.pallas import tpu_sc as plsc

BS = 16
N = 196608
HALF = N // 2
NV = HALF // 16
NBIN = 2048
ONE_F32_BITS = 0x3F800000
IMAX = 0x7FFFFFFF


def _mono(b):
    return b ^ ((b >> 31) & IMAX)


def _zero_hist(hist_v):
    def body(i, carry):
        hist_v[pl.ds(i * 16, 16)] = jnp.zeros((16,), jnp.int32)
        return carry
    lax.fori_loop(0, NBIN // 16, body, 0)


def _hist_scan(hist_v, phist_v, below, target):
    def body(i, carry):
        cum, jstar, less = carry
        h = hist_v[pl.ds(i * 16, 16)] + phist_v[pl.ds(i * 16, 16)]
        for l in range(16):
            hj = h[l]
            cum = cum + hj
            m = (below + cum) < target
            jstar = jstar + jnp.where(m, 1, 0)
            less = jnp.where(m, cum, less)
        return cum, jstar, less

    _, jstar, less = lax.fori_loop(
        0, NBIN // 16, body, (jnp.int32(0), jnp.int32(0), jnp.int32(0)))
    return jstar, below + less


def _combine_hists(hist_v, phist_v, shared, s):
    pltpu.sync_copy(hist_v, shared.at[s])
    plsc.subcore_barrier()
    pltpu.sync_copy(shared.at[s ^ 1], phist_v)
    plsc.subcore_barrier()


def _sc_body(x_hbm, pi_hbm, pf_hbm, out_hbm,
             data_v, hist_v, phist_v, pi_v, pf_v, xch_v, shared):
    c = lax.axis_index("c")
    s = lax.axis_index("s")
    wid = c * 16 + s
    base = wid * HALF

    pltpu.sync_copy(x_hbm.at[pl.ds(base, HALF)], data_v)
    pltpu.sync_copy(pi_hbm, pi_v)
    pltpu.sync_copy(pf_hbm, pf_v)

    piv = pi_v[...]
    pfv = pf_v[...]
    k1 = piv[0]
    zflag = piv[1]
    frac = pfv[0]
    target = k1 + 1

    ones = jnp.ones((16,), jnp.int32)

    _zero_hist(hist_v)

    def p1(i, carry):
        sl = pl.ds(i * 16, 16)
        b = data_v[sl]
        sv = _mono(b)
        data_v[sl] = sv
        plsc.addupdate_scatter(hist_v, [(sv >> 21) + 1024], ones)
        return carry

    lax.fori_loop(0, NV, p1, 0)
    _combine_hists(hist_v, phist_v, shared, s)
    sel1, below = _hist_scan(hist_v, phist_v, jnp.int32(0), target)
    t11 = sel1 - 1024

    _zero_hist(hist_v)

    def p2(i, carry):
        sl = pl.ds(i * 16, 16)
        sv = data_v[sl]
        m = (sv >> 21) == t11
        plsc.addupdate_scatter(hist_v, [(sv >> 10) & 0x7FF], ones, mask=m)
        return carry

    lax.fori_loop(0, NV, p2, 0)
    _combine_hists(hist_v, phist_v, shared, s)
    sel2, below = _hist_scan(hist_v, phist_v, below, target)
    p21 = (t11 << 11) | sel2

    _zero_hist(hist_v)

    def p3(i, carry):
        sl = pl.ds(i * 16, 16)
        sv = data_v[sl]
        m = (sv >> 10) == p21
        plsc.addupdate_scatter(hist_v, [sv & 0x3FF], ones, mask=m)
        return carry

    lax.fori_loop(0, NV, p3, 0)
    _combine_hists(hist_v, phist_v, shared, s)
    sel3, _ = _hist_scan(hist_v, phist_v, below, target)
    v1 = (p21 << 10) | sel3

    def p4(i, carry):
        cnt_v, mn_v = carry
        sv = data_v[pl.ds(i * 16, 16)]
        cnt_v = cnt_v + jnp.where(sv <= v1, 1, 0)
        mn_v = jnp.where(sv > v1, jnp.minimum(mn_v, sv), mn_v)
        return cnt_v, mn_v

    cnt_v, mn_v = lax.fori_loop(
        0, NV, p4,
        (jnp.zeros((16,), jnp.int32), jnp.full((16,), IMAX, jnp.int32)))
    cnt = jnp.int32(0)
    mn = jnp.int32(IMAX)
    for l in range(16):
        cnt = cnt + cnt_v[l]
        mn = jnp.minimum(mn, mn_v[l])

    lanes = lax.iota(jnp.int32, 16)
    xch_v[...] = jnp.where(lanes == 0, cnt, jnp.where(lanes == 1, mn, IMAX))
    pltpu.sync_copy(xch_v, shared.at[s, pl.ds(0, 16)])
    plsc.subcore_barrier()
    pltpu.sync_copy(shared.at[s ^ 1, pl.ds(0, 16)], xch_v)
    plsc.subcore_barrier()
    pv = xch_v[...]
    cnt_t = cnt + pv[0]
    mn_t = jnp.minimum(mn, pv[1])
    v2 = jnp.where(cnt_t >= k1 + 2, v1, mn_t)

    f1 = lax.bitcast_convert_type(_mono(v1), jnp.float32)
    f2 = lax.bitcast_convert_type(_mono(v2), jnp.float32)
    q = f1 * (1.0 - frac) + f2 * frac
    sq = _mono(lax.bitcast_convert_type(q, jnp.int32))
    sq = jnp.where(zflag != 0, jnp.int32(IMAX), sq)

    one_bits = jnp.full((16,), ONE_F32_BITS, jnp.int32)
    zero_bits = jnp.zeros((16,), jnp.int32)

    def p5(i, carry):
        sl = pl.ds(i * 16, 16)
        sv = data_v[sl]
        data_v[sl] = jnp.where(sv >= sq, one_bits, zero_bits)
        return carry

    lax.fori_loop(0, NV, p5, 0)
    pltpu.sync_copy(data_v, out_hbm.at[pl.ds(base, HALF)])


_sc_call = functools.partial(
    pl.kernel,
    out_type=jax.ShapeDtypeStruct((BS * N,), jnp.int32),
    mesh=plsc.VectorSubcoreMesh(core_axis_name="c", subcore_axis_name="s"),
    compiler_params=pltpu.CompilerParams(needs_layout_passes=False),
    scratch_types=[
        pltpu.VMEM((HALF,), jnp.int32),
        pltpu.VMEM((NBIN,), jnp.int32),
        pltpu.VMEM((NBIN,), jnp.int32),
        pltpu.VMEM((16,), jnp.int32),
        pltpu.VMEM((16,), jnp.float32),
        pltpu.VMEM((16,), jnp.int32),
        pltpu.VMEM_SHARED((16, NBIN), jnp.int32),
    ],
)(_sc_body)


def kernel(scale, pr):
    bs, ch, w, h = scale.shape
    n = ch * w * h
    p = jnp.where(pr > 10, 10, pr) * 0.1
    pr_bis = (1.0 - p).astype(jnp.float32)
    idxf = pr_bis * (n - 1)
    k1 = jnp.clip(jnp.floor(idxf), 0, n - 1).astype(jnp.int32)
    frac = idxf - k1.astype(jnp.float32)
    z = (pr == 0).astype(jnp.int32)
    pi = jnp.zeros((16,), jnp.int32).at[0].set(k1).at[1].set(z)
    pf = jnp.zeros((16,), jnp.float32).at[0].set(frac)
    xi = lax.bitcast_convert_type(scale, jnp.int32).reshape(bs * n)
    out = _sc_call(xi, pi, pf)
    return lax.bitcast_convert_type(out.reshape(bs, ch, w, h), jnp.float32)


# speedup vs baseline: 17.3166x; 1.9467x over previous
import functools

import jax
import jax.numpy as jnp
from jax import lax
from jax.experimental import pallas as pl
from jax.experimental.pallas import tpu as pltpu
from jax.experimental.pallas import tpu_sc as plsc

BS = 16
N = 196608
HALF = N // 2
NV = HALF // 16
NBIN = 2048
ONE_F32_BITS = 0x3F800000
IMAX = 0x7FFFFFFF


def _mono(b):
    return b ^ ((b >> 31) & IMAX)


def _zero_hist(hist_v):
    @plsc.parallel_loop(0, NBIN // 16, unroll=8)
    def _(i):
        hist_v[pl.ds(i * 16, 16)] = jnp.zeros((16,), jnp.int32)


def _hist_scan(hist_v, phist_v, below, target):
    def body(i, carry):
        run, jstar, bsel, found = carry
        h = hist_v[pl.ds(i * 16, 16)] + phist_v[pl.ds(i * 16, 16)]
        c = plsc.cumsum(h)
        tot = c[15]
        m = (below + run + c) < target
        nm = plsc.all_reduce_population_count(m)[0]
        cross = (below + run + tot) >= target
        first = jnp.logical_and(cross, found == 0)
        bsel = jnp.where(first, run, bsel)
        found = jnp.where(cross, 1, found)
        return run + tot, jstar + nm, bsel, found

    _, jstar, bsel, _ = lax.fori_loop(
        0, NBIN // 16, body,
        (jnp.int32(0), jnp.int32(0), jnp.int32(0), jnp.int32(0)))
    ci = jstar >> 4
    lane = jstar & 15
    h = hist_v[pl.ds(ci * 16, 16)] + phist_v[pl.ds(ci * 16, 16)]
    hm = jnp.where(lax.iota(jnp.int32, 16) < lane, h, 0)
    less_in = plsc.cumsum(hm)[15]
    return jstar, below + bsel + less_in


def _combine_hists(hist_v, phist_v, shared, s):
    pltpu.sync_copy(hist_v, shared.at[s])
    plsc.subcore_barrier()
    pltpu.sync_copy(shared.at[s ^ 1], phist_v)
    plsc.subcore_barrier()


def _sc_body(x_hbm, pi_hbm, pf_hbm, out_hbm,
             data_v, hist_v, phist_v, pi_v, pf_v, xch_v, shared):
    c = lax.axis_index("c")
    s = lax.axis_index("s")
    wid = c * 16 + s
    base = wid * HALF

    pltpu.sync_copy(x_hbm.at[pl.ds(base, HALF)], data_v)
    pltpu.sync_copy(pi_hbm, pi_v)
    pltpu.sync_copy(pf_hbm, pf_v)

    piv = pi_v[...]
    pfv = pf_v[...]
    k1 = piv[0]
    zflag = piv[1]
    frac = pfv[0]
    target = k1 + 1

    ones = jnp.ones((16,), jnp.int32)

    _zero_hist(hist_v)

    @plsc.parallel_loop(0, NV, unroll=8)
    def p1(i):
        sl = pl.ds(i * 16, 16)
        b = data_v[sl]
        sv = _mono(b)
        data_v[sl] = sv
        plsc.addupdate_scatter(hist_v, [(sv >> 21) + 1024], ones)
    _combine_hists(hist_v, phist_v, shared, s)
    sel1, below = _hist_scan(hist_v, phist_v, jnp.int32(0), target)
    t11 = sel1 - 1024

    _zero_hist(hist_v)

    @plsc.parallel_loop(0, NV, unroll=8)
    def p2(i):
        sv = data_v[pl.ds(i * 16, 16)]
        m = (sv >> 21) == t11
        plsc.addupdate_scatter(hist_v, [(sv >> 10) & 0x7FF], ones, mask=m)
    _combine_hists(hist_v, phist_v, shared, s)
    sel2, below = _hist_scan(hist_v, phist_v, below, target)
    p21 = (t11 << 11) | sel2

    _zero_hist(hist_v)

    @plsc.parallel_loop(0, NV, unroll=8)
    def p3(i):
        sv = data_v[pl.ds(i * 16, 16)]
        m = (sv >> 10) == p21
        plsc.addupdate_scatter(hist_v, [sv & 0x3FF], ones, mask=m)
    _combine_hists(hist_v, phist_v, shared, s)
    sel3, _ = _hist_scan(hist_v, phist_v, below, target)
    v1 = (p21 << 10) | sel3

    @plsc.parallel_loop(
        0, NV, unroll=8,
        carry=(jnp.zeros((16,), jnp.int32), jnp.full((16,), IMAX, jnp.int32)))
    def p4(i, carry):
        cnt_v, mn_v = carry
        sv = data_v[pl.ds(i * 16, 16)]
        cnt_v = cnt_v + jnp.where(sv <= v1, 1, 0)
        mn_v = jnp.where(sv > v1, jnp.minimum(mn_v, sv), mn_v)
        return cnt_v, mn_v

    cnt_v, mn_v = p4
    cnt = plsc.cumsum(cnt_v)[15]
    mn = -plsc.cummax(-mn_v)[15]

    lanes = lax.iota(jnp.int32, 16)
    xch_v[...] = jnp.where(lanes == 0, cnt, jnp.where(lanes == 1, mn, IMAX))
    pltpu.sync_copy(xch_v, shared.at[s, pl.ds(0, 16)])
    plsc.subcore_barrier()
    pltpu.sync_copy(shared.at[s ^ 1, pl.ds(0, 16)], xch_v)
    plsc.subcore_barrier()
    pv = xch_v[...]
    cnt_t = cnt + pv[0]
    mn_t = jnp.minimum(mn, pv[1])
    v2 = jnp.where(cnt_t >= k1 + 2, v1, mn_t)

    f1 = lax.bitcast_convert_type(_mono(v1), jnp.float32)
    f2 = lax.bitcast_convert_type(_mono(v2), jnp.float32)
    q = f1 * (1.0 - frac) + f2 * frac
    sq = _mono(lax.bitcast_convert_type(q, jnp.int32))
    sq = jnp.where(zflag != 0, jnp.int32(IMAX), sq)

    one_bits = jnp.full((16,), ONE_F32_BITS, jnp.int32)
    zero_bits = jnp.zeros((16,), jnp.int32)

    @plsc.parallel_loop(0, NV, unroll=8)
    def p5(i):
        sl = pl.ds(i * 16, 16)
        sv = data_v[sl]
        data_v[sl] = jnp.where(sv >= sq, one_bits, zero_bits)
    pltpu.sync_copy(data_v, out_hbm.at[pl.ds(base, HALF)])


_sc_call = functools.partial(
    pl.kernel,
    out_type=jax.ShapeDtypeStruct((BS * N,), jnp.int32),
    mesh=plsc.VectorSubcoreMesh(core_axis_name="c", subcore_axis_name="s"),
    compiler_params=pltpu.CompilerParams(needs_layout_passes=False),
    scratch_types=[
        pltpu.VMEM((HALF,), jnp.int32),
        pltpu.VMEM((NBIN,), jnp.int32),
        pltpu.VMEM((NBIN,), jnp.int32),
        pltpu.VMEM((16,), jnp.int32),
        pltpu.VMEM((16,), jnp.float32),
        pltpu.VMEM((16,), jnp.int32),
        pltpu.VMEM_SHARED((16, NBIN), jnp.int32),
    ],
)(_sc_body)


def kernel(scale, pr):
    bs, ch, w, h = scale.shape
    n = ch * w * h
    p = jnp.where(pr > 10, 10, pr) * 0.1
    pr_bis = (1.0 - p).astype(jnp.float32)
    idxf = pr_bis * (n - 1)
    k1 = jnp.clip(jnp.floor(idxf), 0, n - 1).astype(jnp.int32)
    frac = idxf - k1.astype(jnp.float32)
    z = (pr == 0).astype(jnp.int32)
    pi = jnp.zeros((16,), jnp.int32).at[0].set(k1).at[1].set(z)
    pf = jnp.zeros((16,), jnp.float32).at[0].set(frac)
    xi = lax.bitcast_convert_type(scale, jnp.int32).reshape(bs * n)
    out = _sc_call(xi, pi, pf)
    return lax.bitcast_convert_type(out.reshape(bs, ch, w, h), jnp.float32)


# speedup vs baseline: 17.9320x; 1.0355x over previous
import functools

import jax
import jax.numpy as jnp
from jax import lax
from jax.experimental import pallas as pl
from jax.experimental.pallas import tpu as pltpu
from jax.experimental.pallas import tpu_sc as plsc

BS = 16
N = 196608
HALF = N // 2
NV = HALF // 16
NBIN = 2048
ONE_F32_BITS = 0x3F800000
IMAX = 0x7FFFFFFF


def _mono(b):
    return b ^ ((b >> 31) & IMAX)


def _zero_hist(hist_v):
    @plsc.parallel_loop(0, NBIN // 16, unroll=8)
    def _(i):
        hist_v[pl.ds(i * 16, 16)] = jnp.zeros((16,), jnp.int32)


def _hist_scan(hist_v, phist_v, below, target, extras=False):
    def body(i, carry):
        run, jstar, bsel, found, nb = carry
        h = hist_v[pl.ds(i * 16, 16)] + phist_v[pl.ds(i * 16, 16)]
        c = plsc.cumsum(h)
        tot = c[15]
        m = (below + run + c) < target
        nm = plsc.all_reduce_population_count(m)[0]
        cross = (below + run + tot) >= target
        first = jnp.logical_and(cross, found == 0)
        bsel = jnp.where(first, run, bsel)
        found = jnp.where(cross, 1, found)
        if extras:
            idxv = i * 16 + lax.iota(jnp.int32, 16)
            past = jnp.logical_and((below + run + (c - h)) >= target, h > 0)
            cand = jnp.where(past, idxv, IMAX)
            nb = jnp.minimum(nb, -plsc.cummax(-cand)[15])
        return run + tot, jstar + nm, bsel, found, nb

    _, jstar, bsel, _, nb = lax.fori_loop(
        0, NBIN // 16, body,
        (jnp.int32(0), jnp.int32(0), jnp.int32(0), jnp.int32(0),
         jnp.int32(IMAX)))
    ci = jstar >> 4
    lane = jstar & 15
    h = hist_v[pl.ds(ci * 16, 16)] + phist_v[pl.ds(ci * 16, 16)]
    lanes = lax.iota(jnp.int32, 16)
    hm = jnp.where(lanes < lane, h, 0)
    less_in = plsc.cumsum(hm)[15]
    if extras:
        hsel = plsc.cumsum(jnp.where(lanes == lane, h, 0))[15]
        return jstar, below + bsel + less_in, nb, hsel
    return jstar, below + bsel + less_in


def _combine_hists(hist_v, phist_v, shared, s):
    pltpu.sync_copy(hist_v, shared.at[s])
    plsc.subcore_barrier()
    pltpu.sync_copy(shared.at[s ^ 1], phist_v)
    plsc.subcore_barrier()


def _sc_body(x_hbm, pi_hbm, pf_hbm, out_hbm,
             data_v, hist_v, phist_v, pi_v, pf_v, xch_v, shared):
    c = lax.axis_index("c")
    s = lax.axis_index("s")
    wid = c * 16 + s
    base = wid * HALF

    pltpu.sync_copy(x_hbm.at[pl.ds(base, HALF)], data_v)
    pltpu.sync_copy(pi_hbm, pi_v)
    pltpu.sync_copy(pf_hbm, pf_v)

    piv = pi_v[...]
    pfv = pf_v[...]
    k1 = piv[0]
    zflag = piv[1]
    frac = pfv[0]
    target = k1 + 1

    ones = jnp.ones((16,), jnp.int32)

    _zero_hist(hist_v)

    @plsc.parallel_loop(0, NV, unroll=8)
    def p1(i):
        sl = pl.ds(i * 16, 16)
        b = data_v[sl]
        sv = _mono(b)
        data_v[sl] = sv
        plsc.addupdate_scatter(hist_v, [(sv >> 21) + 1024], ones)
    _combine_hists(hist_v, phist_v, shared, s)
    sel1, below = _hist_scan(hist_v, phist_v, jnp.int32(0), target)
    t11 = sel1 - 1024

    _zero_hist(hist_v)

    @plsc.parallel_loop(0, NV, unroll=8)
    def p2(i):
        sv = data_v[pl.ds(i * 16, 16)]
        m = (sv >> 21) == t11
        plsc.addupdate_scatter(hist_v, [(sv >> 10) & 0x7FF], ones, mask=m)
    _combine_hists(hist_v, phist_v, shared, s)
    sel2, below = _hist_scan(hist_v, phist_v, below, target)
    p21 = (t11 << 11) | sel2

    _zero_hist(hist_v)

    @plsc.parallel_loop(0, NV, unroll=8,
                        carry=(jnp.full((16,), IMAX, jnp.int32),))
    def p3(i, carry):
        (mn2_v,) = carry
        sv = data_v[pl.ds(i * 16, 16)]
        m = (sv >> 10) == p21
        plsc.addupdate_scatter(hist_v, [sv & 0x3FF], ones, mask=m)
        mn2_v = jnp.where((sv >> 10) > p21, jnp.minimum(mn2_v, sv), mn2_v)
        return (mn2_v,)

    (mn2_v,) = p3
    mn2 = -plsc.cummax(-mn2_v)[15]
    _combine_hists(hist_v, phist_v, shared, s)
    sel3, below3, nb3, hsel3 = _hist_scan(
        hist_v, phist_v, below, target, extras=True)
    v1 = (p21 << 10) | sel3

    lanes = lax.iota(jnp.int32, 16)
    xch_v[...] = jnp.where(lanes == 0, mn2, IMAX)
    pltpu.sync_copy(xch_v, shared.at[s, pl.ds(0, 16)])
    plsc.subcore_barrier()
    pltpu.sync_copy(shared.at[s ^ 1, pl.ds(0, 16)], xch_v)
    plsc.subcore_barrier()
    mn2_t = jnp.minimum(mn2, xch_v[...][0])

    cnt_t = below3 + hsel3
    v2_in = (p21 << 10) | nb3
    v2 = jnp.where(cnt_t >= k1 + 2, v1,
                   jnp.where(nb3 != IMAX, v2_in, mn2_t))

    f1 = lax.bitcast_convert_type(_mono(v1), jnp.float32)
    f2 = lax.bitcast_convert_type(_mono(v2), jnp.float32)
    q = f1 * (1.0 - frac) + f2 * frac
    sq = _mono(lax.bitcast_convert_type(q, jnp.int32))
    sq = jnp.where(zflag != 0, jnp.int32(IMAX), sq)

    one_bits = jnp.full((16,), ONE_F32_BITS, jnp.int32)
    zero_bits = jnp.zeros((16,), jnp.int32)

    @plsc.parallel_loop(0, NV, unroll=8)
    def p5(i):
        sl = pl.ds(i * 16, 16)
        sv = data_v[sl]
        data_v[sl] = jnp.where(sv >= sq, one_bits, zero_bits)
    pltpu.sync_copy(data_v, out_hbm.at[pl.ds(base, HALF)])


_sc_call = functools.partial(
    pl.kernel,
    out_type=jax.ShapeDtypeStruct((BS * N,), jnp.int32),
    mesh=plsc.VectorSubcoreMesh(core_axis_name="c", subcore_axis_name="s"),
    compiler_params=pltpu.CompilerParams(needs_layout_passes=False),
    scratch_types=[
        pltpu.VMEM((HALF,), jnp.int32),
        pltpu.VMEM((NBIN,), jnp.int32),
        pltpu.VMEM((NBIN,), jnp.int32),
        pltpu.VMEM((16,), jnp.int32),
        pltpu.VMEM((16,), jnp.float32),
        pltpu.VMEM((16,), jnp.int32),
        pltpu.VMEM_SHARED((16, NBIN), jnp.int32),
    ],
)(_sc_body)


def kernel(scale, pr):
    bs, ch, w, h = scale.shape
    n = ch * w * h
    p = jnp.where(pr > 10, 10, pr) * 0.1
    pr_bis = (1.0 - p).astype(jnp.float32)
    idxf = pr_bis * (n - 1)
    k1 = jnp.clip(jnp.floor(idxf), 0, n - 1).astype(jnp.int32)
    frac = idxf - k1.astype(jnp.float32)
    z = (pr == 0).astype(jnp.int32)
    pi = jnp.zeros((16,), jnp.int32).at[0].set(k1).at[1].set(z)
    pf = jnp.zeros((16,), jnp.float32).at[0].set(frac)
    xi = lax.bitcast_convert_type(scale, jnp.int32).reshape(bs * n)
    out = _sc_call(xi, pi, pf)
    return lax.bitcast_convert_type(out.reshape(bs, ch, w, h), jnp.float32)
